# baseline (speedup 1.0000x reference)
import functools

import jax
import jax.numpy as jnp
from jax import lax
from jax.experimental import pallas as pl
from jax.experimental.pallas import tpu as pltpu
from jax.experimental.pallas import tpu_sc as plsc

N = 10000
R = 24
D = 256
DH = 128
E = 160000
B = 65536

NC = 2
NS = 16
CH = 128

EPT = 10240
E_PAD = NS * EPT
NCHUNK = EPT // CH
HR_ROWS = (R + 1) * N
DEG_PER_TILE = 16000
DEG_SZ = NS * DEG_PER_TILE
CHD = 2000
AGG_ROWS = N + 64
RPT = 640
RBLK = 40
QPT = B // (NC * NS)
QCHUNK = QPT // CH

_MESH = plsc.VectorSubcoreMesh(core_axis_name="c", subcore_axis_name="s")


def _mm_body(xh_ref, xl_ref, wh_ref, wl_ref, o_ref):
    xh = xh_ref[...]
    xl = xl_ref[...]
    wh = wh_ref[0]
    wl = wl_ref[0]
    o_ref[...] = (
        jnp.dot(xh, wh, preferred_element_type=jnp.float32)
        + jnp.dot(xh, wl, preferred_element_type=jnp.float32)
        + jnp.dot(xl, wh, preferred_element_type=jnp.float32))


def _transform(xh, xl, wh, wl):
    MB = 2000
    nb = N // MB
    return pl.pallas_call(
        _mm_body,
        grid=(nb, R + 1, NC),
        in_specs=[
            pl.BlockSpec((MB, D), lambda i, r, c: (i, 0)),
            pl.BlockSpec((MB, D), lambda i, r, c: (i, 0)),
            pl.BlockSpec((1, D, DH), lambda i, r, c: (r, 0, c)),
            pl.BlockSpec((1, D, DH), lambda i, r, c: (r, 0, c)),
        ],
        out_specs=pl.BlockSpec(
            (MB, DH), lambda i, r, c: (c * (R + 1) * nb + r * nb + i, 0)),
        out_shape=jax.ShapeDtypeStruct((NC * HR_ROWS, DH), jnp.float32),
    )(xh, xl, wh, wl)


def _pdeg_body(t_dst, t_rel, inv_out, deg_s, relb, dstb, keyb, onesb, degb,
               s_load, sem):
    cid = lax.axis_index("c")
    tid = lax.axis_index("s")
    ebase = tid * EPT

    def _zero(k, carry):
        degb[pl.ds(k * 16, 16)] = jnp.zeros((16,), jnp.float32)
        return carry
    lax.fori_loop(0, CHD // 16, _zero, 0)
    def _zcp(k, carry):
        pltpu.sync_copy(degb,
                        deg_s.at[pl.ds(tid * DEG_PER_TILE + k * CHD, CHD)])
        return carry
    lax.fori_loop(0, DEG_PER_TILE // CHD, _zcp, 0)
    for i in range(CH // 16):
        onesb[pl.ds(i * 16, 16)] = jnp.full((16,), 1.0, jnp.float32)
    plsc.subcore_barrier()

    def _fire_loads(j, slot):
        base = ebase + j * CH
        pltpu.async_copy(t_rel.at[pl.ds(base, CH)], relb.at[slot], s_load)
        pltpu.async_copy(t_dst.at[pl.ds(base, CH)], dstb.at[slot], s_load)

    def _wait_loads(slot):
        pltpu.make_async_copy(t_rel.at[pl.ds(0, CH)], relb.at[slot],
                              s_load).wait()
        pltpu.make_async_copy(t_dst.at[pl.ds(0, CH)], dstb.at[slot],
                              s_load).wait()

    def _keys(slot):
        for i in range(CH // 16):
            s = pl.ds(i * 16, 16)
            keyb[slot, s] = relb[slot, s] * N + dstb[slot, s]

    def _wait_scat(slot):
        pltpu.make_async_copy(onesb, deg_s.at[keyb.at[slot]], sem).wait()

    _fire_loads(0, 0)
    _wait_loads(0)
    _keys(0)
    _fire_loads(1, 1)

    def _deg_step(j, carry):
        slot = lax.rem(j, 2)
        nslot = lax.rem(j + 1, 2)
        @pl.when(j + 1 < NCHUNK)
        def _prep():
            @pl.when(j >= 1)
            def _drain():
                _wait_scat(nslot)
            _wait_loads(nslot)
            _keys(nslot)
            @pl.when(j + 2 < NCHUNK)
            def _pref():
                _fire_loads(j + 2, slot)
        pltpu.async_copy(onesb, deg_s.at[keyb.at[slot]], sem, add=True)
        return carry
    lax.fori_loop(0, NCHUNK, _deg_step, 0)
    _wait_scat(0)
    _wait_scat(1)
    plsc.subcore_barrier()

    @pl.when(cid == 0)
    def _write():
        def _invchunk(k, carry):
            off = tid * DEG_PER_TILE + k * CHD
            pltpu.sync_copy(deg_s.at[pl.ds(off, CHD)], degb)
            def _inv(kk, c2):
                v = degb[pl.ds(kk * 16, 16)]
                degb[pl.ds(kk * 16, 16)] = 1.0 / jnp.maximum(v, 1.0)
                return c2
            lax.fori_loop(0, CHD // 16, _inv, 0)
            pltpu.sync_copy(degb, inv_out.at[pl.ds(off, CHD)])
            return carry
        lax.fori_loop(0, DEG_PER_TILE // CHD, _invchunk, 0)


def _degcount(td, tr):
    fn = functools.partial(
        pl.kernel,
        out_type=jax.ShapeDtypeStruct((DEG_SZ,), jnp.float32),
        mesh=_MESH,
        scratch_types=[
            pltpu.VMEM_SHARED((DEG_SZ,), jnp.float32),
            pltpu.VMEM((2, CH), jnp.int32),
            pltpu.VMEM((2, CH), jnp.int32),
            pltpu.VMEM((2, CH), jnp.int32),
            pltpu.VMEM((CH,), jnp.float32),
            pltpu.VMEM((CHD,), jnp.float32),
            pltpu.SemaphoreType.DMA,
            pltpu.SemaphoreType.DMA,
        ],
        compiler_params=pltpu.CompilerParams(needs_layout_passes=False),
    )(_pdeg_body)
    return fn(td, tr)


def _pmsg_body(hr, t_src, t_dst, t_rel, invdeg, h_out,
               agg_s, relb, srcb, dstb, keyb, hidxb, dsc, normb, rowsb, hbuf,
               s_load, s_norm, s_rows, s_scat):
    cid = lax.axis_index("c")
    tid = lax.axis_index("s")
    c_off = cid * HR_ROWS
    ebase = tid * EPT

    nblk = jnp.where(tid == NS - 1, (N - (NS - 1) * RPT) // RBLK, RPT // RBLK)
    def _init_blk(k, carry):
        r0 = tid * RPT + k * RBLK
        pltpu.sync_copy(hr.at[pl.ds(c_off + R * N + r0, RBLK)], hbuf)
        pltpu.sync_copy(hbuf, agg_s.at[pl.ds(r0, RBLK)])
        return carry
    lax.fori_loop(0, nblk, _init_blk, 0)
    plsc.subcore_barrier()

    def _fire_loads(j, slot):
        base = ebase + j * CH
        pltpu.async_copy(t_rel.at[pl.ds(base, CH)], relb.at[slot], s_load)
        pltpu.async_copy(t_dst.at[pl.ds(base, CH)], dstb.at[slot], s_load)
        pltpu.async_copy(t_src.at[pl.ds(base, CH)], srcb.at[slot], s_load)

    def _wait_loads(slot):
        pltpu.make_async_copy(t_rel.at[pl.ds(0, CH)], relb.at[slot],
                              s_load).wait()
        pltpu.make_async_copy(t_dst.at[pl.ds(0, CH)], dstb.at[slot],
                              s_load).wait()
        pltpu.make_async_copy(t_src.at[pl.ds(0, CH)], srcb.at[slot],
                              s_load).wait()

    def _keys(slot):
        for i in range(CH // 16):
            s = pl.ds(i * 16, 16)
            rv = relb[slot, s]
            sv = srcb[slot, s]
            keyb[slot, s] = rv * N + dstb[slot, s]
            hidxb[slot, s] = rv * N + sv + c_off
            dsc[slot, s] = dstb[slot, s]

    def _fire_gathers(slot):
        pltpu.async_copy(invdeg.at[keyb.at[slot]], normb.at[slot], s_norm)
        pltpu.async_copy(hr.at[hidxb.at[slot]], rowsb.at[slot], s_rows)

    def _wait_gathers(slot):
        pltpu.make_async_copy(invdeg.at[keyb.at[slot]], normb.at[slot],
                              s_norm).wait()
        pltpu.make_async_copy(hr.at[hidxb.at[slot]], rowsb.at[slot],
                              s_rows).wait()

    def _scale(slot):
        def _row(i, c2):
            nv = plsc.load_gather(
                normb, [jnp.zeros((16,), jnp.int32) + slot,
                        jnp.zeros((16,), jnp.int32) + i])
            for jj in range(DH // 16):
                s = pl.ds(jj * 16, 16)
                rowsb[slot, i, s] = rowsb[slot, i, s] * nv
            return c2
        lax.fori_loop(0, CH, _row, 0)

    def _fire_scatter(slot):
        pltpu.async_copy(rowsb.at[slot], agg_s.at[dsc.at[slot]], s_scat,
                         add=True)

    def _wait_scatter(slot):
        pltpu.make_async_copy(rowsb.at[slot], agg_s.at[dsc.at[slot]],
                              s_scat).wait()

    _fire_loads(0, 0)
    _wait_loads(0)
    _keys(0)
    _fire_gathers(0)
    _fire_loads(1, 1)

    def _step(j, carry):
        slot = lax.rem(j, 2)
        nslot = lax.rem(j + 1, 2)
        @pl.when(j + 1 < NCHUNK)
        def _prep():
            @pl.when(j >= 1)
            def _drain():
                _wait_scatter(nslot)
            _wait_loads(nslot)
            _keys(nslot)
            _fire_gathers(nslot)
            @pl.when(j + 2 < NCHUNK)
            def _pref():
                _fire_loads(j + 2, slot)
        _wait_gathers(slot)
        _scale(slot)
        _fire_scatter(slot)
        return carry
    lax.fori_loop(0, NCHUNK, _step, 0)
    _wait_scatter(0)
    _wait_scatter(1)
    plsc.subcore_barrier()

    def _out_blk(k, carry):
        r0 = tid * RPT + k * RBLK
        pltpu.sync_copy(agg_s.at[pl.ds(r0, RBLK)], hbuf)
        def _relu_row(i, carry2):
            for jj in range(DH // 16):
                v = hbuf[i, pl.ds(jj * 16, 16)]
                hbuf[i, pl.ds(jj * 16, 16)] = jnp.maximum(v, 0.0)
            return carry2
        lax.fori_loop(0, RBLK, _relu_row, 0)
        pltpu.sync_copy(hbuf, h_out.at[pl.ds(cid * N + r0, RBLK)])
        return carry
    lax.fori_loop(0, nblk, _out_blk, 0)


def _msg(hr, ts, td, tr, invdeg):
    fn = functools.partial(
        pl.kernel,
        out_type=jax.ShapeDtypeStruct((NC * N, DH), jnp.float32),
        mesh=_MESH,
        scratch_types=[
            pltpu.VMEM_SHARED((AGG_ROWS, DH), jnp.float32),
            pltpu.VMEM((2, CH), jnp.int32),
            pltpu.VMEM((2, CH), jnp.int32),
            pltpu.VMEM((2, CH), jnp.int32),
            pltpu.VMEM((2, CH), jnp.int32),
            pltpu.VMEM((2, CH), jnp.int32),
            pltpu.VMEM((2, CH), jnp.int32),
            pltpu.VMEM((2, CH), jnp.float32),
            pltpu.VMEM((2, CH, DH), jnp.float32),
            pltpu.VMEM((RBLK, DH), jnp.float32),
            pltpu.SemaphoreType.DMA,
            pltpu.SemaphoreType.DMA,
            pltpu.SemaphoreType.DMA,
            pltpu.SemaphoreType.DMA,
        ],
        compiler_params=pltpu.CompilerParams(needs_layout_passes=False),
    )(_pmsg_body)
    return fn(hr, ts, td, tr, invdeg)


def _p3_body(h, qs, qo, qr, rel2, scores,
             qsb, qob, qrb, idxb, s0b, s1b, o0b, o1b, relrb, scoreb, sem):
    cid = lax.axis_index("c")
    tid = lax.axis_index("s")
    wid = tid * NC + cid
    qbase = wid * QPT
    lane = lax.broadcasted_iota(jnp.int32, (16,), 0)

    def _chunk(j, carry):
        base = qbase + j * CH
        pltpu.sync_copy(qs.at[pl.ds(base, CH)], qsb.at[0])
        pltpu.sync_copy(qo.at[pl.ds(base, CH)], qob.at[0])
        pltpu.sync_copy(qr.at[pl.ds(base, CH)], qrb)
        for i in range(CH // 16):
            idxb[0, pl.ds(i * 16, 16)] = qsb[0, pl.ds(i * 16, 16)] + N
            idxb[1, pl.ds(i * 16, 16)] = qob[0, pl.ds(i * 16, 16)] + N
        cp1 = pltpu.async_copy(h.at[qsb.at[0]], s0b, sem)
        cp2 = pltpu.async_copy(h.at[qob.at[0]], o0b, sem)
        cp3 = pltpu.async_copy(h.at[idxb.at[0]], s1b, sem)
        cp4 = pltpu.async_copy(h.at[idxb.at[1]], o1b, sem)
        cp5 = pltpu.async_copy(rel2.at[qrb], relrb, sem)
        cp1.wait()
        cp2.wait()
        cp3.wait()
        cp4.wait()
        cp5.wait()

        def _group(g, carry2):
            def _q16(q16, svec):
                q = g * 16 + q16
                acc = jnp.zeros((16,), jnp.float32)
                for jj in range(DH // 16):
                    s = pl.ds(jj * 16, 16)
                    acc = acc + (s0b[q, s] * o0b[q, s] * relrb[q, s])
                for jj in range(DH // 16):
                    s = pl.ds(jj * 16, 16)
                    s2 = pl.ds(DH + jj * 16, 16)
                    acc = acc + (s1b[q, s] * o1b[q, s] * relrb[q, s2])
                sc = jnp.sum(acc)
                return jnp.where(lane == q16, sc, svec)
            svec = lax.fori_loop(0, 16, _q16, jnp.zeros((16,), jnp.float32))
            scoreb[pl.ds(g * 16, 16)] = svec
            return carry2
        lax.fori_loop(0, CH // 16, _group, 0)
        pltpu.sync_copy(scoreb, scores.at[pl.ds(base, CH)])
        return carry
    lax.fori_loop(0, QCHUNK, _chunk, 0)


def _decode(h, qs, qo, qr, rel2):
    fn = functools.partial(
        pl.kernel,
        out_type=jax.ShapeDtypeStruct((B,), jnp.float32),
        mesh=_MESH,
        scratch_types=[
            pltpu.VMEM((1, CH), jnp.int32),
            pltpu.VMEM((1, CH), jnp.int32),
            pltpu.VMEM((CH,), jnp.int32),
            pltpu.VMEM((2, CH), jnp.int32),
            pltpu.VMEM((CH, DH), jnp.float32),
            pltpu.VMEM((CH, DH), jnp.float32),
            pltpu.VMEM((CH, DH), jnp.float32),
            pltpu.VMEM((CH, DH), jnp.float32),
            pltpu.VMEM((CH, D), jnp.float32),
            pltpu.VMEM((CH,), jnp.float32),
            pltpu.SemaphoreType.DMA,
        ],
        compiler_params=pltpu.CompilerParams(needs_layout_passes=False),
    )(_p3_body)
    return fn(h, qs, qo, qr, rel2)


def kernel(triples, train_triples, node_emb, W_rel, W_self, rel_emb):
    qs = triples[:, 0].astype(jnp.int32)
    qo = triples[:, 1].astype(jnp.int32)
    qr = triples[:, 2].astype(jnp.int32)
    ts = train_triples[:, 0].astype(jnp.int32)
    td = train_triples[:, 1].astype(jnp.int32)
    tr = train_triples[:, 2].astype(jnp.int32)
    pad = E_PAD - E
    k = jnp.arange(pad, dtype=jnp.int32)
    ts = jnp.concatenate([ts, k % N])
    td = jnp.concatenate([td, N + (k % 64)])
    tr = jnp.concatenate([tr, jnp.full((pad,), R, jnp.int32)])

    w_big = jnp.concatenate([W_rel, W_self[None]], axis=0)
    xh = node_emb.astype(jnp.bfloat16)
    xl = (node_emb - xh.astype(jnp.float32)).astype(jnp.bfloat16)
    wh = w_big.astype(jnp.bfloat16)
    wl = (w_big - wh.astype(jnp.float32)).astype(jnp.bfloat16)
    invdeg = _degcount(td, tr)
    hr = _transform(xh, xl, wh, wl)
    h = _msg(hr, ts, td, tr, invdeg)
    scores = _decode(h, qs, qo, qr, rel_emb)
    return scores.reshape(B, 1)


# speedup vs baseline: 1.1259x; 1.1259x over previous
import functools

import jax
import jax.numpy as jnp
from jax import lax
from jax.experimental import pallas as pl
from jax.experimental.pallas import tpu as pltpu
from jax.experimental.pallas import tpu_sc as plsc

N = 10000
R = 24
D = 256
DH = 128
E = 160000
B = 65536

NC = 2
NS = 16
CH = 128

EPT = 10240
E_PAD = NS * EPT
NCHUNK = EPT // CH
HR_ROWS = (R + 1) * N
DEG_PER_TILE = 16000
DEG_SZ = NS * DEG_PER_TILE
CHD = 2000
AGG_ROWS = N + 64
RPT = 640
RBLK = 40
QPT = B // (NC * NS)
DCH = 64
DQCHUNK = QPT // DCH

_MESH = plsc.VectorSubcoreMesh(core_axis_name="c", subcore_axis_name="s")


def _mm_body(x_ref, w_ref, o_ref):
    o_ref[...] = jnp.dot(x_ref[...], w_ref[0],
                         preferred_element_type=jnp.float32)


def _transform(x, w_big):
    MB = 2000
    nb = N // MB
    return pl.pallas_call(
        _mm_body,
        grid=(nb, R + 1, NC),
        in_specs=[
            pl.BlockSpec((MB, D), lambda i, r, c: (i, 0)),
            pl.BlockSpec((1, D, DH), lambda i, r, c: (r, 0, c)),
        ],
        out_specs=pl.BlockSpec(
            (MB, DH), lambda i, r, c: (c * (R + 1) * nb + r * nb + i, 0)),
        out_shape=jax.ShapeDtypeStruct((NC * HR_ROWS, DH), jnp.float32),
    )(x, w_big)


def _pdeg_body(t_dst, t_rel, inv_out, deg_s, relb, dstb, keyb, onesb, degb,
               s_load, sem):
    cid = lax.axis_index("c")
    tid = lax.axis_index("s")
    ebase = tid * EPT

    def _zero(k, carry):
        degb[pl.ds(k * 16, 16)] = jnp.zeros((16,), jnp.float32)
        return carry
    lax.fori_loop(0, CHD // 16, _zero, 0)
    def _zcp(k, carry):
        pltpu.sync_copy(degb,
                        deg_s.at[pl.ds(tid * DEG_PER_TILE + k * CHD, CHD)])
        return carry
    lax.fori_loop(0, DEG_PER_TILE // CHD, _zcp, 0)
    for i in range(CH // 16):
        onesb[pl.ds(i * 16, 16)] = jnp.full((16,), 1.0, jnp.float32)
    plsc.subcore_barrier()

    def _fire_loads(j, slot):
        base = ebase + j * CH
        pltpu.async_copy(t_rel.at[pl.ds(base, CH)], relb.at[slot], s_load)
        pltpu.async_copy(t_dst.at[pl.ds(base, CH)], dstb.at[slot], s_load)

    def _wait_loads(slot):
        pltpu.make_async_copy(t_rel.at[pl.ds(0, CH)], relb.at[slot],
                              s_load).wait()
        pltpu.make_async_copy(t_dst.at[pl.ds(0, CH)], dstb.at[slot],
                              s_load).wait()

    def _keys(slot):
        for i in range(CH // 16):
            s = pl.ds(i * 16, 16)
            keyb[slot, s] = relb[slot, s] * N + dstb[slot, s]

    def _wait_scat(slot):
        pltpu.make_async_copy(onesb, deg_s.at[keyb.at[slot]], sem).wait()

    _fire_loads(0, 0)
    _wait_loads(0)
    _keys(0)
    _fire_loads(1, 1)

    def _deg_step(j, carry):
        slot = lax.rem(j, 2)
        nslot = lax.rem(j + 1, 2)
        @pl.when(j + 1 < NCHUNK)
        def _prep():
            @pl.when(j >= 1)
            def _drain():
                _wait_scat(nslot)
            _wait_loads(nslot)
            _keys(nslot)
            @pl.when(j + 2 < NCHUNK)
            def _pref():
                _fire_loads(j + 2, slot)
        pltpu.async_copy(onesb, deg_s.at[keyb.at[slot]], sem, add=True)
        return carry
    lax.fori_loop(0, NCHUNK, _deg_step, 0)
    _wait_scat(0)
    _wait_scat(1)
    plsc.subcore_barrier()

    @pl.when(cid == 0)
    def _write():
        def _invchunk(k, carry):
            off = tid * DEG_PER_TILE + k * CHD
            pltpu.sync_copy(deg_s.at[pl.ds(off, CHD)], degb)
            def _inv(kk, c2):
                v = degb[pl.ds(kk * 16, 16)]
                degb[pl.ds(kk * 16, 16)] = 1.0 / jnp.maximum(v, 1.0)
                return c2
            lax.fori_loop(0, CHD // 16, _inv, 0)
            pltpu.sync_copy(degb, inv_out.at[pl.ds(off, CHD)])
            return carry
        lax.fori_loop(0, DEG_PER_TILE // CHD, _invchunk, 0)


def _degcount(td, tr):
    fn = functools.partial(
        pl.kernel,
        out_type=jax.ShapeDtypeStruct((DEG_SZ,), jnp.float32),
        mesh=_MESH,
        scratch_types=[
            pltpu.VMEM_SHARED((DEG_SZ,), jnp.float32),
            pltpu.VMEM((2, CH), jnp.int32),
            pltpu.VMEM((2, CH), jnp.int32),
            pltpu.VMEM((2, CH), jnp.int32),
            pltpu.VMEM((CH,), jnp.float32),
            pltpu.VMEM((CHD,), jnp.float32),
            pltpu.SemaphoreType.DMA,
            pltpu.SemaphoreType.DMA,
        ],
        compiler_params=pltpu.CompilerParams(needs_layout_passes=False),
    )(_pdeg_body)
    return fn(td, tr)


def _pmsg_body(hr, t_src, t_dst, t_rel, invdeg, h_out,
               agg_s, relb, srcb, dstb, keyb, hidxb, dsc, normb, rowsb, hbuf,
               s_load, s_norm, s_rows, s_scat):
    cid = lax.axis_index("c")
    tid = lax.axis_index("s")
    c_off = cid * HR_ROWS
    ebase = tid * EPT

    nblk = jnp.where(tid == NS - 1, (N - (NS - 1) * RPT) // RBLK, RPT // RBLK)
    def _init_blk(k, carry):
        r0 = tid * RPT + k * RBLK
        pltpu.sync_copy(hr.at[pl.ds(c_off + R * N + r0, RBLK)], hbuf)
        pltpu.sync_copy(hbuf, agg_s.at[pl.ds(r0, RBLK)])
        return carry
    lax.fori_loop(0, nblk, _init_blk, 0)
    plsc.subcore_barrier()

    def _fire_loads(j, slot):
        base = ebase + j * CH
        pltpu.async_copy(t_rel.at[pl.ds(base, CH)], relb.at[slot], s_load)
        pltpu.async_copy(t_dst.at[pl.ds(base, CH)], dstb.at[slot], s_load)
        pltpu.async_copy(t_src.at[pl.ds(base, CH)], srcb.at[slot], s_load)

    def _wait_loads(slot):
        pltpu.make_async_copy(t_rel.at[pl.ds(0, CH)], relb.at[slot],
                              s_load).wait()
        pltpu.make_async_copy(t_dst.at[pl.ds(0, CH)], dstb.at[slot],
                              s_load).wait()
        pltpu.make_async_copy(t_src.at[pl.ds(0, CH)], srcb.at[slot],
                              s_load).wait()

    def _keys(slot):
        for i in range(CH // 16):
            s = pl.ds(i * 16, 16)
            rv = relb[slot, s]
            sv = srcb[slot, s]
            keyb[slot, s] = rv * N + dstb[slot, s]
            hidxb[slot, s] = rv * N + sv + c_off
            dsc[slot, s] = dstb[slot, s]

    def _fire_gathers(slot):
        pltpu.async_copy(invdeg.at[keyb.at[slot]], normb.at[slot], s_norm)
        pltpu.async_copy(hr.at[hidxb.at[slot]], rowsb.at[slot], s_rows)

    def _wait_gathers(slot):
        pltpu.make_async_copy(invdeg.at[keyb.at[slot]], normb.at[slot],
                              s_norm).wait()
        pltpu.make_async_copy(hr.at[hidxb.at[slot]], rowsb.at[slot],
                              s_rows).wait()

    def _scale(slot):
        def _row(i, c2):
            nv = plsc.load_gather(
                normb, [jnp.zeros((16,), jnp.int32) + slot,
                        jnp.zeros((16,), jnp.int32) + i])
            for jj in range(DH // 16):
                s = pl.ds(jj * 16, 16)
                rowsb[slot, i, s] = rowsb[slot, i, s] * nv
            return c2
        lax.fori_loop(0, CH, _row, 0)

    def _fire_scatter(slot):
        pltpu.async_copy(rowsb.at[slot], agg_s.at[dsc.at[slot]], s_scat,
                         add=True)

    def _wait_scatter(slot):
        pltpu.make_async_copy(rowsb.at[slot], agg_s.at[dsc.at[slot]],
                              s_scat).wait()

    _fire_loads(0, 0)
    _wait_loads(0)
    _keys(0)
    _fire_gathers(0)
    _fire_loads(1, 1)

    def _step(j, carry):
        slot = lax.rem(j, 2)
        nslot = lax.rem(j + 1, 2)
        @pl.when(j + 1 < NCHUNK)
        def _prep():
            @pl.when(j >= 1)
            def _drain():
                _wait_scatter(nslot)
            _wait_loads(nslot)
            _keys(nslot)
            _fire_gathers(nslot)
            @pl.when(j + 2 < NCHUNK)
            def _pref():
                _fire_loads(j + 2, slot)
        _wait_gathers(slot)
        _scale(slot)
        _fire_scatter(slot)
        return carry
    lax.fori_loop(0, NCHUNK, _step, 0)
    _wait_scatter(0)
    _wait_scatter(1)
    plsc.subcore_barrier()

    def _out_blk(k, carry):
        r0 = tid * RPT + k * RBLK
        pltpu.sync_copy(agg_s.at[pl.ds(r0, RBLK)], hbuf)
        def _relu_row(i, carry2):
            for jj in range(DH // 16):
                v = hbuf[i, pl.ds(jj * 16, 16)]
                hbuf[i, pl.ds(jj * 16, 16)] = jnp.maximum(v, 0.0)
            return carry2
        lax.fori_loop(0, RBLK, _relu_row, 0)
        pltpu.sync_copy(hbuf, h_out.at[pl.ds(cid * N + r0, RBLK)])
        return carry
    lax.fori_loop(0, nblk, _out_blk, 0)


def _msg(hr, ts, td, tr, invdeg):
    fn = functools.partial(
        pl.kernel,
        out_type=jax.ShapeDtypeStruct((NC * N, DH), jnp.float32),
        mesh=_MESH,
        scratch_types=[
            pltpu.VMEM_SHARED((AGG_ROWS, DH), jnp.float32),
            pltpu.VMEM((2, CH), jnp.int32),
            pltpu.VMEM((2, CH), jnp.int32),
            pltpu.VMEM((2, CH), jnp.int32),
            pltpu.VMEM((2, CH), jnp.int32),
            pltpu.VMEM((2, CH), jnp.int32),
            pltpu.VMEM((2, CH), jnp.int32),
            pltpu.VMEM((2, CH), jnp.float32),
            pltpu.VMEM((2, CH, DH), jnp.float32),
            pltpu.VMEM((RBLK, DH), jnp.float32),
            pltpu.SemaphoreType.DMA,
            pltpu.SemaphoreType.DMA,
            pltpu.SemaphoreType.DMA,
            pltpu.SemaphoreType.DMA,
        ],
        compiler_params=pltpu.CompilerParams(needs_layout_passes=False),
    )(_pmsg_body)
    return fn(hr, ts, td, tr, invdeg)


def _p3_body(h, qs, qo, qr, rel2, scores,
             qsb, qob, qrb, idx1b, idx2b, s0b, s1b, o0b, o1b, relrb, scoreb,
             s_gat):
    cid = lax.axis_index("c")
    tid = lax.axis_index("s")
    wid = tid * NC + cid
    qbase = wid * QPT
    lane = lax.broadcasted_iota(jnp.int32, (16,), 0)

    def _load_and_fire(j, slot):
        base = qbase + j * DCH
        pltpu.sync_copy(qs.at[pl.ds(base, DCH)], qsb.at[slot])
        pltpu.sync_copy(qo.at[pl.ds(base, DCH)], qob.at[slot])
        pltpu.sync_copy(qr.at[pl.ds(base, DCH)], qrb.at[slot])
        for i in range(DCH // 16):
            s = pl.ds(i * 16, 16)
            idx1b[slot, s] = qsb[slot, s] + N
            idx2b[slot, s] = qob[slot, s] + N
        pltpu.async_copy(h.at[qsb.at[slot]], s0b.at[slot], s_gat)
        pltpu.async_copy(h.at[qob.at[slot]], o0b.at[slot], s_gat)
        pltpu.async_copy(h.at[idx1b.at[slot]], s1b.at[slot], s_gat)
        pltpu.async_copy(h.at[idx2b.at[slot]], o1b.at[slot], s_gat)
        pltpu.async_copy(rel2.at[qrb.at[slot]], relrb.at[slot], s_gat)

    def _wait_gathers(slot):
        pltpu.make_async_copy(h.at[qsb.at[slot]], s0b.at[slot], s_gat).wait()
        pltpu.make_async_copy(h.at[qob.at[slot]], o0b.at[slot], s_gat).wait()
        pltpu.make_async_copy(h.at[idx1b.at[slot]], s1b.at[slot],
                              s_gat).wait()
        pltpu.make_async_copy(h.at[idx2b.at[slot]], o1b.at[slot],
                              s_gat).wait()
        pltpu.make_async_copy(rel2.at[qrb.at[slot]], relrb.at[slot],
                              s_gat).wait()

    def _compute(j, slot):
        def _group(g, carry2):
            def _q16(q16, svec):
                q = g * 16 + q16
                acc = jnp.zeros((16,), jnp.float32)
                for jj in range(DH // 16):
                    s = pl.ds(jj * 16, 16)
                    acc = acc + (s0b[slot, q, s] * o0b[slot, q, s]
                                 * relrb[slot, q, s])
                for jj in range(DH // 16):
                    s = pl.ds(jj * 16, 16)
                    s2 = pl.ds(DH + jj * 16, 16)
                    acc = acc + (s1b[slot, q, s] * o1b[slot, q, s]
                                 * relrb[slot, q, s2])
                sc = jnp.sum(acc)
                return jnp.where(lane == q16, sc, svec)
            svec = lax.fori_loop(0, 16, _q16, jnp.zeros((16,), jnp.float32))
            scoreb[slot, pl.ds(g * 16, 16)] = svec
            return carry2
        lax.fori_loop(0, DCH // 16, _group, 0)
        pltpu.sync_copy(scoreb.at[slot],
                        scores.at[pl.ds(qbase + j * DCH, DCH)])

    _load_and_fire(0, 0)
    _load_and_fire(1, 1)

    def _step(j, carry):
        slot = lax.rem(j, 2)
        _wait_gathers(slot)
        _compute(j, slot)
        @pl.when(j + 2 < DQCHUNK)
        def _next():
            _load_and_fire(j + 2, slot)
        return carry
    lax.fori_loop(0, DQCHUNK, _step, 0)


def _decode(h, qs, qo, qr, rel2):
    fn = functools.partial(
        pl.kernel,
        out_type=jax.ShapeDtypeStruct((B,), jnp.float32),
        mesh=_MESH,
        scratch_types=[
            pltpu.VMEM((2, DCH), jnp.int32),
            pltpu.VMEM((2, DCH), jnp.int32),
            pltpu.VMEM((2, DCH), jnp.int32),
            pltpu.VMEM((2, DCH), jnp.int32),
            pltpu.VMEM((2, DCH), jnp.int32),
            pltpu.VMEM((2, DCH, DH), jnp.float32),
            pltpu.VMEM((2, DCH, DH), jnp.float32),
            pltpu.VMEM((2, DCH, DH), jnp.float32),
            pltpu.VMEM((2, DCH, DH), jnp.float32),
            pltpu.VMEM((2, DCH, D), jnp.float32),
            pltpu.VMEM((2, DCH), jnp.float32),
            pltpu.SemaphoreType.DMA,
        ],
        compiler_params=pltpu.CompilerParams(needs_layout_passes=False),
    )(_p3_body)
    return fn(h, qs, qo, qr, rel2)


def kernel(triples, train_triples, node_emb, W_rel, W_self, rel_emb):
    qs = triples[:, 0].astype(jnp.int32)
    qo = triples[:, 1].astype(jnp.int32)
    qr = triples[:, 2].astype(jnp.int32)
    ts = train_triples[:, 0].astype(jnp.int32)
    td = train_triples[:, 1].astype(jnp.int32)
    tr = train_triples[:, 2].astype(jnp.int32)
    pad = E_PAD - E
    k = jnp.arange(pad, dtype=jnp.int32)
    ts = jnp.concatenate([ts, k % N])
    td = jnp.concatenate([td, N + (k % 64)])
    tr = jnp.concatenate([tr, jnp.full((pad,), R, jnp.int32)])

    w_big = jnp.concatenate([W_rel, W_self[None]], axis=0)
    invdeg = _degcount(td, tr)
    hr = _transform(node_emb, w_big)
    h = _msg(hr, ts, td, tr, invdeg)
    scores = _decode(h, qs, qo, qr, rel_emb)
    return scores.reshape(B, 1)


# speedup vs baseline: 1.2410x; 1.1022x over previous
import functools

import jax
import jax.numpy as jnp
from jax import lax
from jax.experimental import pallas as pl
from jax.experimental.pallas import tpu as pltpu
from jax.experimental.pallas import tpu_sc as plsc

N = 10000
R = 24
D = 256
DH = 128
E = 160000
B = 65536

NC = 2
NS = 16
CH = 128

EPT = 10240
E_PAD = NS * EPT
NCHUNK = EPT // CH
HR_ROWS = (R + 1) * N
DEG_PER_TILE = 16000
DEG_SZ = NS * DEG_PER_TILE
CHD = 2000
AGG_ROWS = N + 64
RPT = 640
RBLK = 40
QPT = B // (NC * NS)
DCH = 64
DQCHUNK = QPT // DCH

_MESH = plsc.VectorSubcoreMesh(core_axis_name="c", subcore_axis_name="s")


def _mm_body(x_ref, w_ref, o_ref):
    o_ref[...] = jnp.dot(x_ref[...], w_ref[0],
                         preferred_element_type=jnp.float32)


def _transform(x, w_big):
    MB = 2000
    nb = N // MB
    return pl.pallas_call(
        _mm_body,
        grid=(nb, R + 1, NC),
        in_specs=[
            pl.BlockSpec((MB, D), lambda i, r, c: (i, 0)),
            pl.BlockSpec((1, D, DH), lambda i, r, c: (r, 0, c)),
        ],
        out_specs=pl.BlockSpec(
            (MB, DH), lambda i, r, c: (c * (R + 1) * nb + r * nb + i, 0)),
        out_shape=jax.ShapeDtypeStruct((NC * HR_ROWS, DH), jnp.float32),
    )(x, w_big)


def _pdeg_body(t_dst, t_rel, inv_out, deg_s, relb, dstb, keyb, onesb, degb,
               s_load, sem):
    cid = lax.axis_index("c")
    tid = lax.axis_index("s")
    ebase = tid * EPT

    def _zero(k, carry):
        degb[pl.ds(k * 16, 16)] = jnp.zeros((16,), jnp.float32)
        return carry
    lax.fori_loop(0, CHD // 16, _zero, 0)
    def _zcp(k, carry):
        pltpu.sync_copy(degb,
                        deg_s.at[pl.ds(tid * DEG_PER_TILE + k * CHD, CHD)])
        return carry
    lax.fori_loop(0, DEG_PER_TILE // CHD, _zcp, 0)
    for i in range(CH // 16):
        onesb[pl.ds(i * 16, 16)] = jnp.full((16,), 1.0, jnp.float32)
    plsc.subcore_barrier()

    def _fire_loads(j, slot):
        base = ebase + j * CH
        pltpu.async_copy(t_rel.at[pl.ds(base, CH)], relb.at[slot], s_load)
        pltpu.async_copy(t_dst.at[pl.ds(base, CH)], dstb.at[slot], s_load)

    def _wait_loads(slot):
        pltpu.make_async_copy(t_rel.at[pl.ds(0, CH)], relb.at[slot],
                              s_load).wait()
        pltpu.make_async_copy(t_dst.at[pl.ds(0, CH)], dstb.at[slot],
                              s_load).wait()

    def _keys(slot):
        for i in range(CH // 16):
            s = pl.ds(i * 16, 16)
            keyb[slot, s] = relb[slot, s] * N + dstb[slot, s]

    def _wait_scat(slot):
        pltpu.make_async_copy(onesb, deg_s.at[keyb.at[slot]], sem).wait()

    _fire_loads(0, 0)
    _wait_loads(0)
    _keys(0)
    _fire_loads(1, 1)

    def _deg_step(j, carry):
        slot = lax.rem(j, 2)
        nslot = lax.rem(j + 1, 2)
        @pl.when(j + 1 < NCHUNK)
        def _prep():
            @pl.when(j >= 1)
            def _drain():
                _wait_scat(nslot)
            _wait_loads(nslot)
            _keys(nslot)
            @pl.when(j + 2 < NCHUNK)
            def _pref():
                _fire_loads(j + 2, slot)
        pltpu.async_copy(onesb, deg_s.at[keyb.at[slot]], sem, add=True)
        return carry
    lax.fori_loop(0, NCHUNK, _deg_step, 0)
    _wait_scat(0)
    _wait_scat(1)
    plsc.subcore_barrier()

    @pl.when(cid == 0)
    def _write():
        def _invchunk(k, carry):
            off = tid * DEG_PER_TILE + k * CHD
            pltpu.sync_copy(deg_s.at[pl.ds(off, CHD)], degb)
            def _inv(kk, c2):
                v = degb[pl.ds(kk * 16, 16)]
                degb[pl.ds(kk * 16, 16)] = 1.0 / jnp.maximum(v, 1.0)
                return c2
            lax.fori_loop(0, CHD // 16, _inv, 0)
            pltpu.sync_copy(degb, inv_out.at[pl.ds(off, CHD)])
            return carry
        lax.fori_loop(0, DEG_PER_TILE // CHD, _invchunk, 0)


def _degcount(td, tr):
    fn = functools.partial(
        pl.kernel,
        out_type=jax.ShapeDtypeStruct((DEG_SZ,), jnp.float32),
        mesh=_MESH,
        scratch_types=[
            pltpu.VMEM_SHARED((DEG_SZ,), jnp.float32),
            pltpu.VMEM((2, CH), jnp.int32),
            pltpu.VMEM((2, CH), jnp.int32),
            pltpu.VMEM((2, CH), jnp.int32),
            pltpu.VMEM((CH,), jnp.float32),
            pltpu.VMEM((CHD,), jnp.float32),
            pltpu.SemaphoreType.DMA,
            pltpu.SemaphoreType.DMA,
        ],
        compiler_params=pltpu.CompilerParams(needs_layout_passes=False),
    )(_pdeg_body)
    return fn(td, tr)


def _pmsg_body(hr, t_src, t_dst, t_rel, invdeg, h_out,
               agg_s, relb, srcb, dstb, keyb, hidxb, dsc, normb, rowsb, hbuf,
               s_load, s_norm, s_rows, s_scat):
    cid = lax.axis_index("c")
    tid = lax.axis_index("s")
    c_off = cid * HR_ROWS
    ebase = tid * EPT

    nblk = jnp.where(tid == NS - 1, (N - (NS - 1) * RPT) // RBLK, RPT // RBLK)
    def _init_blk(k, carry):
        r0 = tid * RPT + k * RBLK
        pltpu.sync_copy(hr.at[pl.ds(c_off + R * N + r0, RBLK)], hbuf)
        pltpu.sync_copy(hbuf, agg_s.at[pl.ds(r0, RBLK)])
        return carry
    lax.fori_loop(0, nblk, _init_blk, 0)
    plsc.subcore_barrier()

    def _fire_loads(j, slot):
        base = ebase + j * CH
        pltpu.async_copy(t_rel.at[pl.ds(base, CH)], relb.at[slot], s_load)
        pltpu.async_copy(t_dst.at[pl.ds(base, CH)], dstb.at[slot], s_load)
        pltpu.async_copy(t_src.at[pl.ds(base, CH)], srcb.at[slot], s_load)

    def _wait_loads(slot):
        pltpu.make_async_copy(t_rel.at[pl.ds(0, CH)], relb.at[slot],
                              s_load).wait()
        pltpu.make_async_copy(t_dst.at[pl.ds(0, CH)], dstb.at[slot],
                              s_load).wait()
        pltpu.make_async_copy(t_src.at[pl.ds(0, CH)], srcb.at[slot],
                              s_load).wait()

    def _keys(slot):
        for i in range(CH // 16):
            s = pl.ds(i * 16, 16)
            rv = relb[slot, s]
            sv = srcb[slot, s]
            keyb[slot, s] = rv * N + dstb[slot, s]
            hidxb[slot, s] = rv * N + sv + c_off
            dsc[slot, s] = dstb[slot, s]

    def _fire_gathers(slot):
        pltpu.async_copy(invdeg.at[keyb.at[slot]], normb.at[slot], s_norm)
        pltpu.async_copy(hr.at[hidxb.at[slot]], rowsb.at[slot], s_rows)

    def _wait_gathers(slot):
        pltpu.make_async_copy(invdeg.at[keyb.at[slot]], normb.at[slot],
                              s_norm).wait()
        pltpu.make_async_copy(hr.at[hidxb.at[slot]], rowsb.at[slot],
                              s_rows).wait()

    def _scale(slot):
        def _row(i, c2):
            nv = plsc.load_gather(
                normb, [jnp.zeros((16,), jnp.int32) + slot,
                        jnp.zeros((16,), jnp.int32) + i])
            for jj in range(DH // 16):
                s = pl.ds(jj * 16, 16)
                rowsb[slot, i, s] = rowsb[slot, i, s] * nv
            return c2
        lax.fori_loop(0, CH, _row, 0)

    def _fire_scatter(slot):
        pltpu.async_copy(rowsb.at[slot], agg_s.at[dsc.at[slot]], s_scat,
                         add=True)

    def _wait_scatter(slot):
        pltpu.make_async_copy(rowsb.at[slot], agg_s.at[dsc.at[slot]],
                              s_scat).wait()

    _fire_loads(0, 0)
    _wait_loads(0)
    _keys(0)
    _fire_gathers(0)
    _fire_loads(1, 1)

    def _step(j, carry):
        slot = lax.rem(j, 2)
        nslot = lax.rem(j + 1, 2)
        @pl.when(j + 1 < NCHUNK)
        def _prep():
            @pl.when(j >= 1)
            def _drain():
                _wait_scatter(nslot)
            _wait_loads(nslot)
            _keys(nslot)
            _fire_gathers(nslot)
            @pl.when(j + 2 < NCHUNK)
            def _pref():
                _fire_loads(j + 2, slot)
        _wait_gathers(slot)
        _scale(slot)
        _fire_scatter(slot)
        return carry
    lax.fori_loop(0, NCHUNK, _step, 0)
    _wait_scatter(0)
    _wait_scatter(1)
    plsc.subcore_barrier()

    def _out_blk(k, carry):
        r0 = tid * RPT + k * RBLK
        pltpu.sync_copy(agg_s.at[pl.ds(r0, RBLK)], hbuf)
        def _relu_row(i, carry2):
            for jj in range(DH // 16):
                v = hbuf[i, pl.ds(jj * 16, 16)]
                hbuf[i, pl.ds(jj * 16, 16)] = jnp.maximum(v, 0.0)
            return carry2
        lax.fori_loop(0, RBLK, _relu_row, 0)
        pltpu.sync_copy(hbuf, h_out.at[pl.ds(cid * N + r0, RBLK)])
        return carry
    lax.fori_loop(0, nblk, _out_blk, 0)


def _msg(hr, ts, td, tr, invdeg):
    fn = functools.partial(
        pl.kernel,
        out_type=jax.ShapeDtypeStruct((NC * N, DH), jnp.float32),
        mesh=_MESH,
        scratch_types=[
            pltpu.VMEM_SHARED((AGG_ROWS, DH), jnp.float32),
            pltpu.VMEM((2, CH), jnp.int32),
            pltpu.VMEM((2, CH), jnp.int32),
            pltpu.VMEM((2, CH), jnp.int32),
            pltpu.VMEM((2, CH), jnp.int32),
            pltpu.VMEM((2, CH), jnp.int32),
            pltpu.VMEM((2, CH), jnp.int32),
            pltpu.VMEM((2, CH), jnp.float32),
            pltpu.VMEM((2, CH, DH), jnp.float32),
            pltpu.VMEM((RBLK, DH), jnp.float32),
            pltpu.SemaphoreType.DMA,
            pltpu.SemaphoreType.DMA,
            pltpu.SemaphoreType.DMA,
            pltpu.SemaphoreType.DMA,
        ],
        compiler_params=pltpu.CompilerParams(needs_layout_passes=False),
    )(_pmsg_body)
    return fn(hr, ts, td, tr, invdeg)


def _p3_body(h, qs, qo, qr, rel_flat, scores,
             qsb, qob, qrb, idx1b, idx2b, s0b, s1b, o0b, o1b, relv, scoreb,
             s_gat):
    cid = lax.axis_index("c")
    tid = lax.axis_index("s")
    wid = tid * NC + cid
    qbase = wid * QPT
    lane = lax.broadcasted_iota(jnp.int32, (16,), 0)
    pltpu.sync_copy(rel_flat, relv)

    def _load_and_fire(j, slot):
        base = qbase + j * DCH
        pltpu.sync_copy(qs.at[pl.ds(base, DCH)], qsb.at[slot])
        pltpu.sync_copy(qo.at[pl.ds(base, DCH)], qob.at[slot])
        pltpu.sync_copy(qr.at[pl.ds(base, DCH)], qrb.at[slot])
        for i in range(DCH // 16):
            s = pl.ds(i * 16, 16)
            idx1b[slot, s] = qsb[slot, s] + N
            idx2b[slot, s] = qob[slot, s] + N
        pltpu.async_copy(h.at[qsb.at[slot]], s0b.at[slot], s_gat)
        pltpu.async_copy(h.at[qob.at[slot]], o0b.at[slot], s_gat)
        pltpu.async_copy(h.at[idx1b.at[slot]], s1b.at[slot], s_gat)
        pltpu.async_copy(h.at[idx2b.at[slot]], o1b.at[slot], s_gat)

    def _wait_gathers(slot):
        pltpu.make_async_copy(h.at[qsb.at[slot]], s0b.at[slot], s_gat).wait()
        pltpu.make_async_copy(h.at[qob.at[slot]], o0b.at[slot], s_gat).wait()
        pltpu.make_async_copy(h.at[idx1b.at[slot]], s1b.at[slot],
                              s_gat).wait()
        pltpu.make_async_copy(h.at[idx2b.at[slot]], o1b.at[slot],
                              s_gat).wait()

    def _compute(j, slot):
        def _group(g, carry2):
            def _q16(q16, svec):
                q = g * 16 + q16
                rq = plsc.load_gather(
                    qrb, [jnp.zeros((16,), jnp.int32) + slot,
                          jnp.zeros((16,), jnp.int32) + q])
                rbase = rq * D
                acc = jnp.zeros((16,), jnp.float32)
                for jj in range(DH // 16):
                    rv = plsc.load_gather(relv, [rbase + jj * 16 + lane])
                    acc = acc + (s0b[slot, q, pl.ds(jj * 16, 16)]
                                 * o0b[slot, q, pl.ds(jj * 16, 16)] * rv)
                for jj in range(DH // 16):
                    rv = plsc.load_gather(relv, [rbase + DH + jj * 16 + lane])
                    acc = acc + (s1b[slot, q, pl.ds(jj * 16, 16)]
                                 * o1b[slot, q, pl.ds(jj * 16, 16)] * rv)
                sc = jnp.sum(acc)
                return jnp.where(lane == q16, sc, svec)
            svec = lax.fori_loop(0, 16, _q16, jnp.zeros((16,), jnp.float32))
            scoreb[slot, pl.ds(g * 16, 16)] = svec
            return carry2
        lax.fori_loop(0, DCH // 16, _group, 0)
        pltpu.sync_copy(scoreb.at[slot],
                        scores.at[pl.ds(qbase + j * DCH, DCH)])

    _load_and_fire(0, 0)
    _load_and_fire(1, 1)

    def _step(j, carry):
        slot = lax.rem(j, 2)
        _wait_gathers(slot)
        _compute(j, slot)
        @pl.when(j + 2 < DQCHUNK)
        def _next():
            _load_and_fire(j + 2, slot)
        return carry
    lax.fori_loop(0, DQCHUNK, _step, 0)


def _decode(h, qs, qo, qr, rel_flat):
    fn = functools.partial(
        pl.kernel,
        out_type=jax.ShapeDtypeStruct((B,), jnp.float32),
        mesh=_MESH,
        scratch_types=[
            pltpu.VMEM((2, DCH), jnp.int32),
            pltpu.VMEM((2, DCH), jnp.int32),
            pltpu.VMEM((2, DCH), jnp.int32),
            pltpu.VMEM((2, DCH), jnp.int32),
            pltpu.VMEM((2, DCH), jnp.int32),
            pltpu.VMEM((2, DCH, DH), jnp.float32),
            pltpu.VMEM((2, DCH, DH), jnp.float32),
            pltpu.VMEM((2, DCH, DH), jnp.float32),
            pltpu.VMEM((2, DCH, DH), jnp.float32),
            pltpu.VMEM((R * D,), jnp.float32),
            pltpu.VMEM((2, DCH), jnp.float32),
            pltpu.SemaphoreType.DMA,
        ],
        compiler_params=pltpu.CompilerParams(needs_layout_passes=False),
    )(_p3_body)
    return fn(h, qs, qo, qr, rel_flat)


def kernel(triples, train_triples, node_emb, W_rel, W_self, rel_emb):
    qs = triples[:, 0].astype(jnp.int32)
    qo = triples[:, 1].astype(jnp.int32)
    qr = triples[:, 2].astype(jnp.int32)
    ts = train_triples[:, 0].astype(jnp.int32)
    td = train_triples[:, 1].astype(jnp.int32)
    tr = train_triples[:, 2].astype(jnp.int32)
    pad = E_PAD - E
    k = jnp.arange(pad, dtype=jnp.int32)
    ts = jnp.concatenate([ts, k % N])
    td = jnp.concatenate([td, N + (k % 64)])
    tr = jnp.concatenate([tr, jnp.full((pad,), R, jnp.int32)])

    w_big = jnp.concatenate([W_rel, W_self[None]], axis=0)
    invdeg = _degcount(td, tr)
    hr = _transform(node_emb, w_big)
    h = _msg(hr, ts, td, tr, invdeg)
    scores = _decode(h, qs, qo, qr, rel_emb.reshape(-1))
    return scores.reshape(B, 1)
